# baseline (speedup 1.0000x reference)
import functools

import jax
import jax.numpy as jnp
from jax import lax
from jax.experimental import pallas as pl
from jax.experimental.pallas import tpu as pltpu
from jax.experimental.pallas import tpu_sc as plsc

B = 4
S = 4096
D = 768
EPS = 1e-12
NC = 2
NS = 16
NW = NC * NS
POS_PER_W = S // NW
P = 8
TT = P * B
NSUPER = POS_PER_W // P
NVREG = D // 16
NG = 8


def _lane_allreduce_sum(x):
    lanes = lax.iota(jnp.int32, 16)
    for k in (8, 4, 2, 1):
        x = x + x.at[lanes ^ k].get(mode="promise_in_bounds")
    return x


def _finalize(s, q):
    s = _lane_allreduce_sum(s)
    q = _lane_allreduce_sum(q)
    inv_d = jnp.float32(1.0 / D)
    mean_v = s * inv_d
    var_v = q * inv_d - mean_v * mean_v + jnp.float32(EPS)
    i = lax.bitcast_convert_type(var_v, jnp.int32)
    y = lax.bitcast_convert_type(jnp.int32(0x5F3759DF) - (i >> 1),
                                 jnp.float32)
    half_var = var_v * jnp.float32(0.5)
    for _ in range(3):
        y = y * (jnp.float32(1.5) - half_var * y * y)
    return mean_v, y


def _compute_chunk(w_v, pos_v, g_v, b_v, o_v):
    def group_body(gq, c):
        p0 = gq * 2
        p1 = p0 + 1
        rows = [b * P + p0 for b in range(B)] + [b * P + p1 for b in range(B)]

        def p1_(j, acc):
            ds = pl.ds(j * 16, 16)
            pv0 = pos_v[p0, ds]
            pv1 = pos_v[p1, ds]
            sums = list(acc[:NG])
            sqs = list(acc[NG:])
            for k in range(NG):
                x = w_v[rows[k], ds] + (pv0 if k < B else pv1)
                sums[k] = sums[k] + x
                sqs[k] = sqs[k] + x * x
            return tuple(sums) + tuple(sqs)

        z = jnp.zeros((16,), jnp.float32)
        acc = lax.fori_loop(0, NVREG, p1_, (z,) * (2 * NG))
        my = [_finalize(acc[k], acc[NG + k]) for k in range(NG)]

        @plsc.parallel_loop(0, NVREG, unroll=2)
        def p2(j):
            ds = pl.ds(j * 16, 16)
            gv = g_v[ds]
            bv = b_v[ds]
            pv0 = pos_v[p0, ds]
            pv1 = pos_v[p1, ds]
            for k in range(NG):
                mk, yk = my[k]
                tk = yk * gv
                x = w_v[rows[k], ds] + (pv0 if k < B else pv1)
                o_v[rows[k], ds] = (x - mk) * tk + bv

        return c

    lax.fori_loop(0, P // 2, group_body, 0)


def _body(ids_hbm, wt_hbm, pt_hbm, g_hbm, bt_hbm, out_hbm,
          idx_v, g_v, b_v, pos_a, pos_b, w_a, w_b, o_a, o_b,
          psem_a, psem_b, gsem_a, gsem_b, osem_a, osem_b):
    cid = lax.axis_index("c")
    sid = lax.axis_index("s")
    wid = cid * NS + sid
    wpos = wid * POS_PER_W

    pltpu.sync_copy(g_hbm, g_v)
    pltpu.sync_copy(bt_hbm, b_v)
    pltpu.sync_copy(ids_hbm.at[:, pl.ds(wpos, POS_PER_W)], idx_v)

    def idx_slice(s, b):
        return idx_v.at[b, pl.ds(s * P, P)]

    def pos_slice(s):
        return pt_hbm.at[pl.ds(wpos + s * P, P)]

    def gathers(s, w_v, gsem):
        for b in range(B):
            pltpu.async_copy(wt_hbm.at[idx_slice(s, b)],
                             w_v.at[pl.ds(b * P, P)], gsem)

    def wait_gathers(s, w_v, gsem):
        for b in range(B):
            pltpu.make_async_copy(wt_hbm.at[idx_slice(s, b)],
                                  w_v.at[pl.ds(b * P, P)], gsem).wait()

    def outs(s, o_v, osem):
        for b in range(B):
            pltpu.async_copy(o_v.at[pl.ds(b * P, P)],
                             out_hbm.at[b, pl.ds(wpos + s * P, P)], osem)

    def wait_outs(s, o_v, osem):
        for b in range(B):
            pltpu.make_async_copy(o_v.at[pl.ds(b * P, P)],
                                  out_hbm.at[b, pl.ds(wpos + s * P, P)],
                                  osem).wait()

    pltpu.async_copy(pos_slice(0), pos_a, psem_a)
    gathers(0, w_a, gsem_a)

    def chunk(s, w_v, o_v, pos_v, gsem, osem, psem,
              w_nxt, pos_nxt, gsem_nxt, psem_nxt, last):
        wait_gathers(s, w_v, gsem)
        if not last:
            gathers(s + 1, w_nxt, gsem_nxt)
            pltpu.async_copy(pos_slice(s + 1), pos_nxt, psem_nxt)
        pltpu.make_async_copy(pos_slice(s), pos_v, psem).wait()

        @pl.when(s >= 2)
        def _():
            wait_outs(s - 2, o_v, osem)

        _compute_chunk(w_v, pos_v, g_v, b_v, o_v)
        outs(s, o_v, osem)

    def pair(i, carry, last_pair=False):
        ss = i * 2
        chunk(ss, w_a, o_a, pos_a, gsem_a, osem_a, psem_a,
              w_b, pos_b, gsem_b, psem_b, False)
        chunk(ss + 1, w_b, o_b, pos_b, gsem_b, osem_b, psem_b,
              w_a, pos_a, gsem_a, psem_a, last_pair)
        return carry

    lax.fori_loop(0, NSUPER // 2 - 1, pair, 0)
    pair(NSUPER // 2 - 1, 0, last_pair=True)
    wait_outs(NSUPER - 2, o_a, osem_a)
    wait_outs(NSUPER - 1, o_b, osem_b)


@jax.jit
def _run(input_ids, word_table, pos_table, gamma, beta):
    mesh = plsc.VectorSubcoreMesh(core_axis_name="c", subcore_axis_name="s")
    kern = functools.partial(
        pl.kernel,
        out_type=jax.ShapeDtypeStruct((B, S, D), jnp.float32),
        mesh=mesh,
        scratch_types=[
            pltpu.VMEM((B, POS_PER_W), jnp.int32),
            pltpu.VMEM((D,), jnp.float32),
            pltpu.VMEM((D,), jnp.float32),
            pltpu.VMEM((P, D), jnp.float32),
            pltpu.VMEM((P, D), jnp.float32),
            pltpu.VMEM((TT, D), jnp.float32),
            pltpu.VMEM((TT, D), jnp.float32),
            pltpu.VMEM((TT, D), jnp.float32),
            pltpu.VMEM((TT, D), jnp.float32),
            pltpu.SemaphoreType.DMA,
            pltpu.SemaphoreType.DMA,
            pltpu.SemaphoreType.DMA,
            pltpu.SemaphoreType.DMA,
            pltpu.SemaphoreType.DMA,
            pltpu.SemaphoreType.DMA,
        ],
    )(_body)
    return kern(input_ids, word_table, pos_table, gamma, beta)


def kernel(input_ids, word_table, pos_table, gamma, beta):
    ids = input_ids.astype(jnp.int32)
    return _run(ids, word_table, pos_table, gamma, beta)


# speedup vs baseline: 1.1059x; 1.1059x over previous
import functools

import jax
import jax.numpy as jnp
from jax import lax
from jax.experimental import pallas as pl
from jax.experimental.pallas import tpu as pltpu
from jax.experimental.pallas import tpu_sc as plsc

B = 4
S = 4096
D = 768
EPS = 1e-12
NC = 2
NS = 16
NW = NC * NS
POS_PER_W = S // NW
P = 8
TT = P * B
NSUPER = POS_PER_W // P
NVREG = D // 16
NG = 8


def _lane_allreduce_sum(x):
    lanes = lax.iota(jnp.int32, 16)
    for k in (8, 4, 2, 1):
        x = x + x.at[lanes ^ k].get(mode="promise_in_bounds")
    return x


def _finalize(s, q):
    s = _lane_allreduce_sum(s)
    q = _lane_allreduce_sum(q)
    inv_d = jnp.float32(1.0 / D)
    mean_v = s * inv_d
    var_v = q * inv_d - mean_v * mean_v + jnp.float32(EPS)
    i = lax.bitcast_convert_type(var_v, jnp.int32)
    y = lax.bitcast_convert_type(jnp.int32(0x5F3759DF) - (i >> 1),
                                 jnp.float32)
    half_var = var_v * jnp.float32(0.5)
    for _ in range(3):
        y = y * (jnp.float32(1.5) - half_var * y * y)
    return mean_v, y


def _compute_chunk(w_v, pos_v, g_v, b_v, o_v):
    def group_body(gq, c):
        p0 = gq * 2
        p1 = p0 + 1
        rows = [b * P + p0 for b in range(B)] + [b * P + p1 for b in range(B)]

        def p1_(j, acc):
            ds = pl.ds(j * 16, 16)
            pv0 = pos_v[p0, ds]
            pv1 = pos_v[p1, ds]
            sums = list(acc[:NG])
            sqs = list(acc[NG:])
            for k in range(NG):
                x = w_v[rows[k], ds] + (pv0 if k < B else pv1)
                sums[k] = sums[k] + x
                sqs[k] = sqs[k] + x * x
            return tuple(sums) + tuple(sqs)

        z = jnp.zeros((16,), jnp.float32)
        acc = lax.fori_loop(0, NVREG, p1_, (z,) * (2 * NG))
        my = [_finalize(acc[k], acc[NG + k]) for k in range(NG)]

        @plsc.parallel_loop(0, NVREG, unroll=2)
        def p2(j):
            ds = pl.ds(j * 16, 16)
            gv = g_v[ds]
            bv = b_v[ds]
            pv0 = pos_v[p0, ds]
            pv1 = pos_v[p1, ds]
            for k in range(NG):
                mk, yk = my[k]
                tk = yk * gv
                ck = bv - mk * tk
                x = w_v[rows[k], ds] + (pv0 if k < B else pv1)
                o_v[rows[k], ds] = x * tk + ck

        return c

    lax.fori_loop(0, P // 2, group_body, 0)


def _body(ids_hbm, wt_hbm, pt_hbm, g_hbm, bt_hbm, out_hbm,
          idx_v, g_v, b_v, pos_a, pos_b, w_a, w_b, o_a, o_b,
          psem_a, psem_b, gsem_a, gsem_b, osem_a, osem_b):
    cid = lax.axis_index("c")
    sid = lax.axis_index("s")
    wid = cid * NS + sid
    wpos = wid * POS_PER_W

    pltpu.sync_copy(g_hbm, g_v)
    pltpu.sync_copy(bt_hbm, b_v)
    pltpu.sync_copy(ids_hbm.at[:, pl.ds(wpos, POS_PER_W)], idx_v)

    def idx_slice(s, b):
        return idx_v.at[b, pl.ds(s * P, P)]

    def pos_slice(s):
        return pt_hbm.at[pl.ds(wpos + s * P, P)]

    def gathers(s, w_v, gsem):
        for b in range(B):
            pltpu.async_copy(wt_hbm.at[idx_slice(s, b)],
                             w_v.at[pl.ds(b * P, P)], gsem)

    def wait_gathers(s, w_v, gsem):
        for b in range(B):
            pltpu.make_async_copy(wt_hbm.at[idx_slice(s, b)],
                                  w_v.at[pl.ds(b * P, P)], gsem).wait()

    def outs(s, o_v, osem):
        for b in range(B):
            pltpu.async_copy(o_v.at[pl.ds(b * P, P)],
                             out_hbm.at[b, pl.ds(wpos + s * P, P)], osem)

    def wait_outs(s, o_v, osem):
        for b in range(B):
            pltpu.make_async_copy(o_v.at[pl.ds(b * P, P)],
                                  out_hbm.at[b, pl.ds(wpos + s * P, P)],
                                  osem).wait()

    pltpu.async_copy(pos_slice(0), pos_a, psem_a)
    gathers(0, w_a, gsem_a)

    def chunk(s, w_v, o_v, pos_v, gsem, osem, psem,
              w_nxt, pos_nxt, gsem_nxt, psem_nxt, last):
        wait_gathers(s, w_v, gsem)
        if not last:
            gathers(s + 1, w_nxt, gsem_nxt)
            pltpu.async_copy(pos_slice(s + 1), pos_nxt, psem_nxt)
        pltpu.make_async_copy(pos_slice(s), pos_v, psem).wait()

        @pl.when(s >= 2)
        def _():
            wait_outs(s - 2, o_v, osem)

        _compute_chunk(w_v, pos_v, g_v, b_v, o_v)
        outs(s, o_v, osem)

    def pair(i, carry, last_pair=False):
        ss = i * 2
        chunk(ss, w_a, o_a, pos_a, gsem_a, osem_a, psem_a,
              w_b, pos_b, gsem_b, psem_b, False)
        chunk(ss + 1, w_b, o_b, pos_b, gsem_b, osem_b, psem_b,
              w_a, pos_a, gsem_a, psem_a, last_pair)
        return carry

    lax.fori_loop(0, NSUPER // 2 - 1, pair, 0)
    pair(NSUPER // 2 - 1, 0, last_pair=True)
    wait_outs(NSUPER - 2, o_a, osem_a)
    wait_outs(NSUPER - 1, o_b, osem_b)


@jax.jit
def _run(input_ids, word_table, pos_table, gamma, beta):
    mesh = plsc.VectorSubcoreMesh(core_axis_name="c", subcore_axis_name="s")
    kern = functools.partial(
        pl.kernel,
        out_type=jax.ShapeDtypeStruct((B, S, D), jnp.float32),
        mesh=mesh,
        scratch_types=[
            pltpu.VMEM((B, POS_PER_W), jnp.int32),
            pltpu.VMEM((D,), jnp.float32),
            pltpu.VMEM((D,), jnp.float32),
            pltpu.VMEM((P, D), jnp.float32),
            pltpu.VMEM((P, D), jnp.float32),
            pltpu.VMEM((TT, D), jnp.float32),
            pltpu.VMEM((TT, D), jnp.float32),
            pltpu.VMEM((TT, D), jnp.float32),
            pltpu.VMEM((TT, D), jnp.float32),
            pltpu.SemaphoreType.DMA,
            pltpu.SemaphoreType.DMA,
            pltpu.SemaphoreType.DMA,
            pltpu.SemaphoreType.DMA,
            pltpu.SemaphoreType.DMA,
            pltpu.SemaphoreType.DMA,
        ],
    )(_body)
    return kern(input_ids, word_table, pos_table, gamma, beta)


def kernel(input_ids, word_table, pos_table, gamma, beta):
    ids = input_ids.astype(jnp.int32)
    return _run(ids, word_table, pos_table, gamma, beta)


# speedup vs baseline: 1.1196x; 1.0124x over previous
import functools

import jax
import jax.numpy as jnp
from jax import lax
from jax.experimental import pallas as pl
from jax.experimental.pallas import tpu as pltpu
from jax.experimental.pallas import tpu_sc as plsc

B = 4
S = 4096
D = 768
EPS = 1e-12
NC = 2
NS = 16
NW = NC * NS
POS_PER_W = S // NW
P = 8
TT = P * B
NSUPER = POS_PER_W // P
NVREG = D // 16
NG = 8


def _lane_allreduce_sum(x):
    lanes = lax.iota(jnp.int32, 16)
    for k in (8, 4, 2, 1):
        x = x + x.at[lanes ^ k].get(mode="promise_in_bounds")
    return x


def _finalize(s, q):
    s = _lane_allreduce_sum(s)
    q = _lane_allreduce_sum(q)
    inv_d = jnp.float32(1.0 / D)
    mean_v = s * inv_d
    var_v = q * inv_d - mean_v * mean_v + jnp.float32(EPS)
    i = lax.bitcast_convert_type(var_v, jnp.int32)
    y = lax.bitcast_convert_type(jnp.int32(0x5F3759DF) - (i >> 1),
                                 jnp.float32)
    half_var = var_v * jnp.float32(0.5)
    for _ in range(3):
        y = y * (jnp.float32(1.5) - half_var * y * y)
    return mean_v, y


def _compute_chunk(w_v, pos_v, g_v, b_v, o_v):
    def group_body(gq, c):
        p0 = gq * 2
        p1 = p0 + 1
        rows = [b * P + p0 for b in range(B)] + [b * P + p1 for b in range(B)]

        def p1_(j, acc):
            ds = pl.ds(j * 16, 16)
            pv0 = pos_v[p0, ds]
            pv1 = pos_v[p1, ds]
            sums = list(acc[:NG])
            sqs = list(acc[NG:])
            for k in range(NG):
                x = w_v[rows[k], ds] + (pv0 if k < B else pv1)
                sums[k] = sums[k] + x
                sqs[k] = sqs[k] + x * x
            return tuple(sums) + tuple(sqs)

        z = jnp.zeros((16,), jnp.float32)
        acc = lax.fori_loop(0, NVREG, p1_, (z,) * (2 * NG), unroll=2)
        my = [_finalize(acc[k], acc[NG + k]) for k in range(NG)]

        @plsc.parallel_loop(0, NVREG, unroll=2)
        def p2(j):
            ds = pl.ds(j * 16, 16)
            gv = g_v[ds]
            bv = b_v[ds]
            pv0 = pos_v[p0, ds]
            pv1 = pos_v[p1, ds]
            for k in range(NG):
                mk, yk = my[k]
                tk = yk * gv
                ck = bv - mk * tk
                x = w_v[rows[k], ds] + (pv0 if k < B else pv1)
                o_v[rows[k], ds] = x * tk + ck

        return c

    lax.fori_loop(0, P // 2, group_body, 0)


def _body(ids_hbm, wt_hbm, pt_hbm, g_hbm, bt_hbm, out_hbm,
          idx_v, g_v, b_v, pos_a, pos_b, w_a, w_b, o_a, o_b,
          psem_a, psem_b, gsem_a, gsem_b, osem_a, osem_b):
    cid = lax.axis_index("c")
    sid = lax.axis_index("s")
    wid = cid * NS + sid
    wpos = wid * POS_PER_W

    pltpu.sync_copy(g_hbm, g_v)
    pltpu.sync_copy(bt_hbm, b_v)
    pltpu.sync_copy(ids_hbm.at[:, pl.ds(wpos, POS_PER_W)], idx_v)

    def idx_slice(s, b):
        return idx_v.at[b, pl.ds(s * P, P)]

    def pos_slice(s):
        return pt_hbm.at[pl.ds(wpos + s * P, P)]

    def gathers(s, w_v, gsem):
        for b in range(B):
            pltpu.async_copy(wt_hbm.at[idx_slice(s, b)],
                             w_v.at[pl.ds(b * P, P)], gsem)

    def wait_gathers(s, w_v, gsem):
        for b in range(B):
            pltpu.make_async_copy(wt_hbm.at[idx_slice(s, b)],
                                  w_v.at[pl.ds(b * P, P)], gsem).wait()

    def outs(s, o_v, osem):
        for b in range(B):
            pltpu.async_copy(o_v.at[pl.ds(b * P, P)],
                             out_hbm.at[b, pl.ds(wpos + s * P, P)], osem)

    def wait_outs(s, o_v, osem):
        for b in range(B):
            pltpu.make_async_copy(o_v.at[pl.ds(b * P, P)],
                                  out_hbm.at[b, pl.ds(wpos + s * P, P)],
                                  osem).wait()

    pltpu.async_copy(pos_slice(0), pos_a, psem_a)
    gathers(0, w_a, gsem_a)

    def chunk(s, w_v, o_v, pos_v, gsem, osem, psem,
              w_nxt, pos_nxt, gsem_nxt, psem_nxt, last):
        wait_gathers(s, w_v, gsem)
        if not last:
            gathers(s + 1, w_nxt, gsem_nxt)
            pltpu.async_copy(pos_slice(s + 1), pos_nxt, psem_nxt)
        pltpu.make_async_copy(pos_slice(s), pos_v, psem).wait()

        @pl.when(s >= 2)
        def _():
            wait_outs(s - 2, o_v, osem)

        _compute_chunk(w_v, pos_v, g_v, b_v, o_v)
        outs(s, o_v, osem)

    def pair(i, carry, last_pair=False):
        ss = i * 2
        chunk(ss, w_a, o_a, pos_a, gsem_a, osem_a, psem_a,
              w_b, pos_b, gsem_b, psem_b, False)
        chunk(ss + 1, w_b, o_b, pos_b, gsem_b, osem_b, psem_b,
              w_a, pos_a, gsem_a, psem_a, last_pair)
        return carry

    lax.fori_loop(0, NSUPER // 2 - 1, pair, 0)
    pair(NSUPER // 2 - 1, 0, last_pair=True)
    wait_outs(NSUPER - 2, o_a, osem_a)
    wait_outs(NSUPER - 1, o_b, osem_b)


@jax.jit
def _run(input_ids, word_table, pos_table, gamma, beta):
    mesh = plsc.VectorSubcoreMesh(core_axis_name="c", subcore_axis_name="s")
    kern = functools.partial(
        pl.kernel,
        out_type=jax.ShapeDtypeStruct((B, S, D), jnp.float32),
        mesh=mesh,
        scratch_types=[
            pltpu.VMEM((B, POS_PER_W), jnp.int32),
            pltpu.VMEM((D,), jnp.float32),
            pltpu.VMEM((D,), jnp.float32),
            pltpu.VMEM((P, D), jnp.float32),
            pltpu.VMEM((P, D), jnp.float32),
            pltpu.VMEM((TT, D), jnp.float32),
            pltpu.VMEM((TT, D), jnp.float32),
            pltpu.VMEM((TT, D), jnp.float32),
            pltpu.VMEM((TT, D), jnp.float32),
            pltpu.SemaphoreType.DMA,
            pltpu.SemaphoreType.DMA,
            pltpu.SemaphoreType.DMA,
            pltpu.SemaphoreType.DMA,
            pltpu.SemaphoreType.DMA,
            pltpu.SemaphoreType.DMA,
        ],
    )(_body)
    return kern(input_ids, word_table, pos_table, gamma, beta)


def kernel(input_ids, word_table, pos_table, gamma, beta):
    ids = input_ids.astype(jnp.int32)
    return _run(ids, word_table, pos_table, gamma, beta)
